# baseline (speedup 1.0000x reference)
import jax
import jax.numpy as jnp
from jax.experimental import pallas as pl
from jax.experimental.pallas import tpu as pltpu

B, C, N, K = 16, 768, 1024, 2
NB = 4


def _norm_rows(x):
    n = jnp.sqrt(jnp.sum(x * x, axis=1, keepdims=True))
    return x / jnp.maximum(n, 1e-12)


def _body(f_ref, c_ref, centers_ref, labels_ref, labelp_ref, onehot_ref,
          weight_ref, cini_ref):
    step = pl.program_id(0)
    c0n = _norm_rows(c_ref[...])

    @pl.when(step == 0)
    def _init():
        centers_ref[...] = jnp.zeros_like(centers_ref)
        cini_ref[...] = jnp.zeros_like(cini_ref)

    centers_acc = jnp.zeros((K, C), jnp.float32)
    cini_acc = jnp.float32(0.0)
    for i in range(NB):
        F = f_ref[i]
        sumsq = jnp.sum(F * F, axis=0, keepdims=True)
        rinv = 1.0 / jnp.maximum(jnp.sqrt(sumsq), 1e-12)
        Fn16 = (F * rinv).astype(jnp.bfloat16)
        centers_n = c0n
        for Ci in range(6):
            cos = jnp.dot(centers_n.astype(jnp.bfloat16), Fn16,
                          preferred_element_type=jnp.float32)
            d = 0.5 * (1.0 - cos)
            mask0 = d[0:1, :] <= d[1:2, :]
            m0 = mask0.astype(jnp.float32)
            masks = jnp.concatenate([m0, 1.0 - m0], axis=0)
            masksT = jnp.transpose(masks)
            sumsT = jnp.dot(F, masksT,
                            precision=jax.lax.Precision.HIGHEST,
                            preferred_element_type=jnp.float32)
            sums = jnp.transpose(sumsT)
            n0 = jnp.sum(m0, axis=1, keepdims=True)
            counts = jnp.concatenate([n0, N - n0], axis=0)
            centers_new = sums / (counts + 1.0)
            if Ci == 0:
                labelp_ref[i] = jnp.where(mask0, 0, 1).astype(jnp.int32)
                cd = jnp.sum(_norm_rows(centers_new) * c0n, axis=1)
                cini_acc = cini_acc + jnp.mean(cd)
            if Ci == 5:
                labels_ref[i] = jnp.where(mask0, 0, 1).astype(jnp.int32)
                onehot_ref[i] = masks
                dmin = jnp.min(d, axis=1, keepdims=True)
                dmax = jnp.max(d, axis=1, keepdims=True)
                weight_ref[i] = 1.0 - (d - dmin) / (dmax - dmin + 1e-7)
                centers_acc = centers_acc + centers_new
            centers_n = _norm_rows(centers_new)

    centers_ref[...] += centers_acc
    cini_ref[...] += jnp.reshape(cini_acc, (1, 1))


def kernel(FeatureT, centerInit):
    Fr = FeatureT.reshape(B, C, N)
    grid = (B // NB,)
    out = pl.pallas_call(
        _body,
        grid=grid,
        in_specs=[
            pl.BlockSpec((NB, C, N), lambda b: (b, 0, 0)),
            pl.BlockSpec((K, C), lambda b: (0, 0)),
        ],
        out_specs=[
            pl.BlockSpec((K, C), lambda b: (0, 0)),
            pl.BlockSpec((NB, 1, N), lambda b: (b, 0, 0)),
            pl.BlockSpec((NB, 1, N), lambda b: (b, 0, 0)),
            pl.BlockSpec((NB, K, N), lambda b: (b, 0, 0)),
            pl.BlockSpec((NB, K, N), lambda b: (b, 0, 0)),
            pl.BlockSpec((1, 1), lambda b: (0, 0)),
        ],
        out_shape=[
            jax.ShapeDtypeStruct((K, C), jnp.float32),
            jax.ShapeDtypeStruct((B, 1, N), jnp.int32),
            jax.ShapeDtypeStruct((B, 1, N), jnp.int32),
            jax.ShapeDtypeStruct((B, K, N), jnp.float32),
            jax.ShapeDtypeStruct((B, K, N), jnp.float32),
            jax.ShapeDtypeStruct((1, 1), jnp.float32),
        ],
        compiler_params=pltpu.CompilerParams(
            dimension_semantics=("arbitrary",),
        ),
    )(Fr, centerInit)
    centers_sum, labels3, labelp3, onehot_t, weight_t, cini = out
    centersIterout = jax.lax.stop_gradient(centers_sum / B)
    labelsout = labels3.reshape(B, N)
    labelPinit = labelp3.reshape(B, N)
    labels_onehotout = onehot_t.transpose(0, 2, 1)
    Weight = weight_t.transpose(0, 2, 1)
    Cinidist = jax.lax.stop_gradient((cini / B).reshape(()))
    return (centersIterout, labelsout, labels_onehotout, Weight, labelPinit,
            Cinidist)


# speedup vs baseline: 1.4102x; 1.4102x over previous
import jax
import jax.numpy as jnp
from jax.experimental import pallas as pl
from jax.experimental.pallas import tpu as pltpu

B, C, N, K = 16, 768, 1024, 2
NB = 4


def _norm_rows(x):
    n = jnp.sqrt(jnp.sum(x * x, axis=1, keepdims=True))
    return x / jnp.maximum(n, 1e-12)


def _body(f_ref, c_ref, centers_ref, labels_ref, labelp_ref, onehot_ref,
          weight_ref, cini_ref):
    step = pl.program_id(0)
    c0n = _norm_rows(c_ref[...])

    @pl.when(step == 0)
    def _init():
        centers_ref[...] = jnp.zeros_like(centers_ref)
        cini_ref[...] = jnp.zeros_like(cini_ref)

    centers_acc = jnp.zeros((K, C), jnp.float32)
    cini_acc = jnp.float32(0.0)
    Fs, Fn16s, sumAlls = [], [], []
    ones_row = jnp.ones((1, N), jnp.float32)
    for i in range(NB):
        F = f_ref[i]
        sumsq = jnp.sum(F * F, axis=0, keepdims=True)
        rinv = 1.0 / jnp.maximum(jnp.sqrt(sumsq), 1e-12)
        Fn16s.append((F * rinv).astype(jnp.bfloat16))
        Fs.append(F)
        sumAlls.append(jax.lax.dot_general(
            ones_row, F, (((1,), (1,)), ((), ())),
            precision=jax.lax.Precision.HIGHEST,
            preferred_element_type=jnp.float32))
    centers_ns = [c0n] * NB
    for Ci in range(6):
        for i in range(NB):
            F = Fs[i]
            cos = jnp.dot(centers_ns[i].astype(jnp.bfloat16), Fn16s[i],
                          preferred_element_type=jnp.float32)
            d = 0.5 * (1.0 - cos)
            mask0 = d[0:1, :] <= d[1:2, :]
            m0 = mask0.astype(jnp.float32)
            sums0 = jax.lax.dot_general(
                m0, F, (((1,), (1,)), ((), ())),
                precision=jax.lax.Precision.HIGHEST,
                preferred_element_type=jnp.float32)
            sums = jnp.concatenate([sums0, sumAlls[i] - sums0], axis=0)
            n0 = jnp.sum(m0, axis=1, keepdims=True)
            counts = jnp.concatenate([n0, N - n0], axis=0)
            centers_new = sums / (counts + 1.0)
            if Ci == 0:
                labelp_ref[i] = jnp.where(mask0, 0, 1).astype(jnp.int32)
                cd = jnp.sum(_norm_rows(centers_new) * c0n, axis=1)
                cini_acc = cini_acc + jnp.mean(cd)
            if Ci == 5:
                labels_ref[i] = jnp.where(mask0, 0, 1).astype(jnp.int32)
                onehot_ref[i] = jnp.concatenate([m0, 1.0 - m0], axis=0)
                dmin = jnp.min(d, axis=1, keepdims=True)
                dmax = jnp.max(d, axis=1, keepdims=True)
                weight_ref[i] = 1.0 - (d - dmin) / (dmax - dmin + 1e-7)
                centers_acc = centers_acc + centers_new
            centers_ns[i] = _norm_rows(centers_new)

    centers_ref[...] += centers_acc
    cini_ref[...] += jnp.reshape(cini_acc, (1, 1))


def kernel(FeatureT, centerInit):
    Fr = FeatureT.reshape(B, C, N)
    grid = (B // NB,)
    out = pl.pallas_call(
        _body,
        grid=grid,
        in_specs=[
            pl.BlockSpec((NB, C, N), lambda b: (b, 0, 0)),
            pl.BlockSpec((K, C), lambda b: (0, 0)),
        ],
        out_specs=[
            pl.BlockSpec((K, C), lambda b: (0, 0)),
            pl.BlockSpec((NB, 1, N), lambda b: (b, 0, 0)),
            pl.BlockSpec((NB, 1, N), lambda b: (b, 0, 0)),
            pl.BlockSpec((NB, K, N), lambda b: (b, 0, 0)),
            pl.BlockSpec((NB, K, N), lambda b: (b, 0, 0)),
            pl.BlockSpec((1, 1), lambda b: (0, 0)),
        ],
        out_shape=[
            jax.ShapeDtypeStruct((K, C), jnp.float32),
            jax.ShapeDtypeStruct((B, 1, N), jnp.int32),
            jax.ShapeDtypeStruct((B, 1, N), jnp.int32),
            jax.ShapeDtypeStruct((B, K, N), jnp.float32),
            jax.ShapeDtypeStruct((B, K, N), jnp.float32),
            jax.ShapeDtypeStruct((1, 1), jnp.float32),
        ],
        compiler_params=pltpu.CompilerParams(
            dimension_semantics=("arbitrary",),
        ),
    )(Fr, centerInit)
    centers_sum, labels3, labelp3, onehot_t, weight_t, cini = out
    centersIterout = jax.lax.stop_gradient(centers_sum / B)
    labelsout = labels3.reshape(B, N)
    labelPinit = labelp3.reshape(B, N)
    labels_onehotout = onehot_t.transpose(0, 2, 1)
    Weight = weight_t.transpose(0, 2, 1)
    Cinidist = jax.lax.stop_gradient((cini / B).reshape(()))
    return (centersIterout, labelsout, labels_onehotout, Weight, labelPinit,
            Cinidist)


# speedup vs baseline: 1.4185x; 1.0059x over previous
import jax
import jax.numpy as jnp
from jax.experimental import pallas as pl
from jax.experimental.pallas import tpu as pltpu

B, C, N, K = 16, 768, 1024, 2
NB = 4


def _norm_rows(x):
    n = jnp.sqrt(jnp.sum(x * x, axis=1, keepdims=True))
    return x / jnp.maximum(n, 1e-12)


def _body(f_ref, c_ref, centers_ref, labels_ref, labelp_ref, onehot_ref,
          weight_ref, cini_ref):
    step = pl.program_id(0)
    c0n = _norm_rows(c_ref[...])

    @pl.when(step == 0)
    def _init():
        centers_ref[...] = jnp.zeros_like(centers_ref)
        cini_ref[...] = jnp.zeros_like(cini_ref)

    centers_acc = jnp.zeros((K, C), jnp.float32)
    cini_acc = jnp.float32(0.0)
    Fs, Fn16s, sumAlls = [], [], []
    ones_row = jnp.ones((1, N), jnp.float32)
    for i in range(NB):
        F = f_ref[i]
        sumsq = jnp.sum(F * F, axis=0, keepdims=True)
        rinv = 1.0 / jnp.maximum(jnp.sqrt(sumsq), 1e-12)
        Fn16s.append((F * rinv).astype(jnp.bfloat16))
        Fs.append(F)
        sumAlls.append(jax.lax.dot_general(
            ones_row, F, (((1,), (1,)), ((), ())),
            precision=jax.lax.Precision.HIGHEST,
            preferred_element_type=jnp.float32))
    centers_ns = [c0n] * NB
    for Ci in range(6):
        for i in range(NB):
            F = Fs[i]
            cn16 = centers_ns[i].astype(jnp.bfloat16)
            cos = jnp.concatenate(
                [jnp.dot(cn16, Fn16s[i][:, h * (N // 2):(h + 1) * (N // 2)],
                         preferred_element_type=jnp.float32)
                 for h in range(2)], axis=1)
            d = 0.5 * (1.0 - cos)
            mask0 = d[0:1, :] <= d[1:2, :]
            m0 = mask0.astype(jnp.float32)
            sums0 = jnp.concatenate(
                [jax.lax.dot_general(
                    m0, F[h * (C // 2):(h + 1) * (C // 2), :],
                    (((1,), (1,)), ((), ())),
                    precision=jax.lax.Precision.HIGHEST,
                    preferred_element_type=jnp.float32)
                 for h in range(2)], axis=1)
            sums = jnp.concatenate([sums0, sumAlls[i] - sums0], axis=0)
            n0 = jnp.sum(m0, axis=1, keepdims=True)
            counts = jnp.concatenate([n0, N - n0], axis=0)
            centers_new = sums / (counts + 1.0)
            if Ci == 0:
                labelp_ref[i] = jnp.where(mask0, 0, 1).astype(jnp.int32)
                cd = jnp.sum(_norm_rows(centers_new) * c0n, axis=1)
                cini_acc = cini_acc + jnp.mean(cd)
            if Ci == 5:
                labels_ref[i] = jnp.where(mask0, 0, 1).astype(jnp.int32)
                onehot_ref[i] = jnp.concatenate([m0, 1.0 - m0], axis=0)
                dmin = jnp.min(d, axis=1, keepdims=True)
                dmax = jnp.max(d, axis=1, keepdims=True)
                weight_ref[i] = 1.0 - (d - dmin) / (dmax - dmin + 1e-7)
                centers_acc = centers_acc + centers_new
            centers_ns[i] = _norm_rows(centers_new)

    centers_ref[...] += centers_acc
    cini_ref[...] += jnp.reshape(cini_acc, (1, 1))


def kernel(FeatureT, centerInit):
    Fr = FeatureT.reshape(B, C, N)
    grid = (B // NB,)
    out = pl.pallas_call(
        _body,
        grid=grid,
        in_specs=[
            pl.BlockSpec((NB, C, N), lambda b: (b, 0, 0)),
            pl.BlockSpec((K, C), lambda b: (0, 0)),
        ],
        out_specs=[
            pl.BlockSpec((K, C), lambda b: (0, 0)),
            pl.BlockSpec((NB, 1, N), lambda b: (b, 0, 0)),
            pl.BlockSpec((NB, 1, N), lambda b: (b, 0, 0)),
            pl.BlockSpec((NB, K, N), lambda b: (b, 0, 0)),
            pl.BlockSpec((NB, K, N), lambda b: (b, 0, 0)),
            pl.BlockSpec((1, 1), lambda b: (0, 0)),
        ],
        out_shape=[
            jax.ShapeDtypeStruct((K, C), jnp.float32),
            jax.ShapeDtypeStruct((B, 1, N), jnp.int32),
            jax.ShapeDtypeStruct((B, 1, N), jnp.int32),
            jax.ShapeDtypeStruct((B, K, N), jnp.float32),
            jax.ShapeDtypeStruct((B, K, N), jnp.float32),
            jax.ShapeDtypeStruct((1, 1), jnp.float32),
        ],
        compiler_params=pltpu.CompilerParams(
            dimension_semantics=("arbitrary",),
        ),
    )(Fr, centerInit)
    centers_sum, labels3, labelp3, onehot_t, weight_t, cini = out
    centersIterout = jax.lax.stop_gradient(centers_sum / B)
    labelsout = labels3.reshape(B, N)
    labelPinit = labelp3.reshape(B, N)
    labels_onehotout = onehot_t.transpose(0, 2, 1)
    Weight = weight_t.transpose(0, 2, 1)
    Cinidist = jax.lax.stop_gradient((cini / B).reshape(()))
    return (centersIterout, labelsout, labels_onehotout, Weight, labelPinit,
            Cinidist)


# speedup vs baseline: 1.4295x; 1.0077x over previous
import jax
import jax.numpy as jnp
from jax.experimental import pallas as pl
from jax.experimental.pallas import tpu as pltpu

B, C, N, K = 16, 768, 1024, 2
NB = 4


def _norm_rows(x):
    n = jnp.sqrt(jnp.sum(x * x, axis=1, keepdims=True))
    return x / jnp.maximum(n, 1e-12)


def _body(f_ref, c_ref, centers_ref, labels_ref, labelp_ref, onehot_ref,
          weight_ref, cini_ref):
    step = pl.program_id(0)
    c0n = _norm_rows(c_ref[...])

    @pl.when(step == 0)
    def _init():
        centers_ref[...] = jnp.zeros_like(centers_ref)
        cini_ref[...] = jnp.zeros_like(cini_ref)

    centers_acc = jnp.zeros((K, C), jnp.float32)
    cini_acc = jnp.float32(0.0)
    Fs, Fn16s, sumAlls = [], [], []
    ones_row = jnp.ones((1, N), jnp.float32)
    for i in range(NB):
        F = f_ref[i]
        sumsq = jnp.sum(F * F, axis=0, keepdims=True)
        rinv = 1.0 / jnp.maximum(jnp.sqrt(sumsq), 1e-12)
        Fn16s.append((F * rinv).astype(jnp.bfloat16))
        Fs.append(F)
        sumAlls.append(jax.lax.dot_general(
            ones_row, F, (((1,), (1,)), ((), ())),
            precision=jax.lax.Precision.HIGHEST,
            preferred_element_type=jnp.float32))
    centers_ns = [c0n] * NB
    for Ci in range(6):
        for i in range(NB):
            F = Fs[i]
            cn16 = centers_ns[i].astype(jnp.bfloat16)
            cos = jnp.concatenate(
                [jnp.dot(cn16, Fn16s[i][:, h * (N // 2):(h + 1) * (N // 2)],
                         preferred_element_type=jnp.float32)
                 for h in range(2)], axis=1)
            d = 0.5 * (1.0 - cos)
            mask0 = d[0:1, :] <= d[1:2, :]
            m0 = mask0.astype(jnp.float32)
            sums0 = jnp.concatenate(
                [jax.lax.dot_general(
                    m0, F[h * (C // 2):(h + 1) * (C // 2), :],
                    (((1,), (1,)), ((), ())),
                    precision=jax.lax.Precision.HIGHEST,
                    preferred_element_type=jnp.float32)
                 for h in range(2)], axis=1)
            sums = jnp.concatenate([sums0, sumAlls[i] - sums0], axis=0)
            n0 = jnp.sum(m0, axis=1, keepdims=True)
            counts = jnp.concatenate([n0, N - n0], axis=0)
            centers_new = sums / (counts + 1.0)
            if Ci == 0:
                labelp_ref[i] = jnp.where(mask0, 0, 1).astype(jnp.int32)
                cd = jnp.sum(_norm_rows(centers_new) * c0n, axis=1)
                cini_acc = cini_acc + jnp.mean(cd)
            if Ci == 5:
                labels_ref[i] = jnp.where(mask0, 0, 1).astype(jnp.int32)
                onehot_ref[i] = jnp.concatenate([m0, 1.0 - m0], axis=0)
                dmin = jnp.min(d, axis=1, keepdims=True)
                dmax = jnp.max(d, axis=1, keepdims=True)
                weight_ref[i] = 1.0 - (d - dmin) / (dmax - dmin + 1e-7)
                centers_acc = centers_acc + centers_new
            centers_ns[i] = _norm_rows(centers_new)

    centers_ref[...] += centers_acc
    cini_ref[...] += jnp.reshape(cini_acc, (1, 1))
    @pl.when(step == (B // NB) - 1)
    def _finish():
        centers_ref[...] = centers_ref[...] * jnp.float32(1.0 / B)
        cini_ref[...] = cini_ref[...] * jnp.float32(1.0 / B)


def kernel(FeatureT, centerInit):
    Fr = FeatureT.reshape(B, C, N)
    grid = (B // NB,)
    out = pl.pallas_call(
        _body,
        grid=grid,
        in_specs=[
            pl.BlockSpec((NB, C, N), lambda b: (b, 0, 0)),
            pl.BlockSpec((K, C), lambda b: (0, 0)),
        ],
        out_specs=[
            pl.BlockSpec((K, C), lambda b: (0, 0)),
            pl.BlockSpec((NB, 1, N), lambda b: (b, 0, 0)),
            pl.BlockSpec((NB, 1, N), lambda b: (b, 0, 0)),
            pl.BlockSpec((NB, K, N), lambda b: (b, 0, 0)),
            pl.BlockSpec((NB, K, N), lambda b: (b, 0, 0)),
            pl.BlockSpec((1, 1), lambda b: (0, 0)),
        ],
        out_shape=[
            jax.ShapeDtypeStruct((K, C), jnp.float32),
            jax.ShapeDtypeStruct((B, 1, N), jnp.int32),
            jax.ShapeDtypeStruct((B, 1, N), jnp.int32),
            jax.ShapeDtypeStruct((B, K, N), jnp.float32),
            jax.ShapeDtypeStruct((B, K, N), jnp.float32),
            jax.ShapeDtypeStruct((1, 1), jnp.float32),
        ],
        compiler_params=pltpu.CompilerParams(
            dimension_semantics=("arbitrary",),
        ),
    )(Fr, centerInit)
    centers_sum, labels3, labelp3, onehot_t, weight_t, cini = out
    centersIterout = jax.lax.stop_gradient(centers_sum)
    labelsout = labels3.reshape(B, N)
    labelPinit = labelp3.reshape(B, N)
    labels_onehotout = onehot_t.transpose(0, 2, 1)
    Weight = weight_t.transpose(0, 2, 1)
    Cinidist = jax.lax.stop_gradient(cini.reshape(()))
    return (centersIterout, labelsout, labels_onehotout, Weight, labelPinit,
            Cinidist)
